# baseline (speedup 1.0000x reference)
import jax
import jax.numpy as jnp
from jax.experimental import pallas as pl
from jax.experimental.pallas import tpu as pltpu

B = 8192
D = 768
L = 256
K = 8192

BB = 512
LA = L + 1
LANES = 128


def _loss_body(x_ref, a_ref, h_ref, cc_ref, out_ref, cca_ref):
    i = pl.program_id(0)

    @pl.when(i == 0)
    def _():
        cf = cc_ref[...]
        c2 = jnp.sum(cf * cf, axis=1)
        cca_ref[...] = jnp.concatenate(
            [cf, (-0.5 * c2).reshape(K, 1)], axis=1).astype(jnp.bfloat16)

    diff = x_ref[...] - a_ref[...]
    recon = jnp.sum(diff * diff)

    h = h_ref[...]
    h2 = jnp.sum(h * h, axis=1)
    ha = jnp.concatenate(
        [h.astype(jnp.bfloat16), jnp.ones((BB, 1), jnp.bfloat16)], axis=1)

    s = jax.lax.dot_general(
        ha, cca_ref[...],
        (((1,), (1,)), ((), ())),
        preferred_element_type=jnp.float32,
    )

    m = jnp.full((BB, LANES), -jnp.inf, dtype=jnp.float32)
    for t in range(K // LANES):
        m = jnp.maximum(m, s[:, t * LANES:(t + 1) * LANES])
    m_row = jnp.max(m, axis=1)

    d2 = h2 - 2.0 * m_row
    part = jnp.reshape(recon / (B * D) + jnp.sum(d2) / (B * L), (1, 1))

    @pl.when(i == 0)
    def _():
        out_ref[...] = jnp.zeros((1, 1), jnp.float32)
    out_ref[...] += part


def kernel(x, h_x, a_x, cluster_centers):
    out = pl.pallas_call(
        _loss_body,
        grid=(B // BB,),
        in_specs=[
            pl.BlockSpec((BB, D), lambda i: (i, 0)),
            pl.BlockSpec((BB, D), lambda i: (i, 0)),
            pl.BlockSpec((BB, L), lambda i: (i, 0)),
            pl.BlockSpec((K, L), lambda i: (0, 0)),
        ],
        out_specs=pl.BlockSpec((1, 1), lambda i: (0, 0)),
        out_shape=jax.ShapeDtypeStruct((1, 1), jnp.float32),
        scratch_shapes=[pltpu.VMEM((K, LA), jnp.bfloat16)],
    )(x, a_x, h_x, cluster_centers)
    return out[0, 0]


# speedup vs baseline: 1.7303x; 1.7303x over previous
import jax
import jax.numpy as jnp
from jax.experimental import pallas as pl
from jax.experimental.pallas import tpu as pltpu

B = 8192
D = 768
L = 256
K = 8192

BB = 512
LANES = 128


def _loss_body(x_ref, a_ref, h_ref, cc_ref, out_ref, c2_ref, ccb_ref):
    i = pl.program_id(0)

    @pl.when(i == 0)
    def _():
        cf = cc_ref[...]
        c2 = jnp.sum(cf * cf, axis=1)
        c2_ref[...] = (0.5 * c2).reshape(1, K)
        ccb_ref[...] = cf.astype(jnp.bfloat16)

    diff = x_ref[...] - a_ref[...]
    recon = jnp.sum(diff * diff)

    h = h_ref[...]
    h2 = jnp.sum(h * h, axis=1)

    s = jax.lax.dot_general(
        h.astype(jnp.bfloat16), ccb_ref[...],
        (((1,), (1,)), ((), ())),
        preferred_element_type=jnp.float32,
    )

    m = jnp.full((BB, LANES), -jnp.inf, dtype=jnp.float32)
    for t in range(K // LANES):
        sl = slice(t * LANES, (t + 1) * LANES)
        m = jnp.maximum(m, s[:, sl] - c2_ref[0:1, sl])
    m_row = jnp.max(m, axis=1)

    d2 = h2 - 2.0 * m_row
    part = jnp.reshape(recon / (B * D) + jnp.sum(d2) / (B * L), (1, 1))

    @pl.when(i == 0)
    def _():
        out_ref[...] = jnp.zeros((1, 1), jnp.float32)
    out_ref[...] += part


def kernel(x, h_x, a_x, cluster_centers):
    out = pl.pallas_call(
        _loss_body,
        grid=(B // BB,),
        in_specs=[
            pl.BlockSpec((BB, D), lambda i: (i, 0)),
            pl.BlockSpec((BB, D), lambda i: (i, 0)),
            pl.BlockSpec((BB, L), lambda i: (i, 0)),
            pl.BlockSpec((K, L), lambda i: (0, 0)),
        ],
        out_specs=pl.BlockSpec((1, 1), lambda i: (0, 0)),
        out_shape=jax.ShapeDtypeStruct((1, 1), jnp.float32),
        scratch_shapes=[pltpu.VMEM((1, K), jnp.float32),
                        pltpu.VMEM((K, L), jnp.bfloat16)],
    )(x, a_x, h_x, cluster_centers)
    return out[0, 0]


# speedup vs baseline: 1.7568x; 1.0153x over previous
import jax
import jax.numpy as jnp
from jax.experimental import pallas as pl
from jax.experimental.pallas import tpu as pltpu

B = 8192
D = 768
L = 256
K = 8192

BB = 512
LANES = 128


def _loss_body(x_ref, a_ref, h_ref, cc_ref, out_ref, c2_ref, ccb_ref):
    i = pl.program_id(0)

    @pl.when(i == 0)
    def _():
        cf = cc_ref[...]
        c2 = jnp.sum(cf * cf, axis=1)
        c2_ref[...] = (0.5 * c2).reshape(1, K).astype(jnp.bfloat16)
        ccb_ref[...] = cf.astype(jnp.bfloat16)

    diff = x_ref[...] - a_ref[...]
    recon = jnp.sum(diff * diff)

    h = h_ref[...]
    h2 = jnp.sum(h * h, axis=1)

    s = jax.lax.dot_general(
        h.astype(jnp.bfloat16), ccb_ref[...],
        (((1,), (1,)), ((), ())),
        preferred_element_type=jnp.float32,
    )

    sb = s.astype(jnp.bfloat16)
    m = jnp.full((BB, LANES), -jnp.inf, dtype=jnp.bfloat16)
    for t in range(K // LANES):
        sl = slice(t * LANES, (t + 1) * LANES)
        m = jnp.maximum(m, sb[:, sl] - c2_ref[0:1, sl])
    m_row = jnp.max(m.astype(jnp.float32), axis=1)

    d2 = h2 - 2.0 * m_row
    part = jnp.reshape(recon / (B * D) + jnp.sum(d2) / (B * L), (1, 1))

    @pl.when(i == 0)
    def _():
        out_ref[...] = jnp.zeros((1, 1), jnp.float32)
    out_ref[...] += part


def kernel(x, h_x, a_x, cluster_centers):
    out = pl.pallas_call(
        _loss_body,
        grid=(B // BB,),
        in_specs=[
            pl.BlockSpec((BB, D), lambda i: (i, 0)),
            pl.BlockSpec((BB, D), lambda i: (i, 0)),
            pl.BlockSpec((BB, L), lambda i: (i, 0)),
            pl.BlockSpec((K, L), lambda i: (0, 0)),
        ],
        out_specs=pl.BlockSpec((1, 1), lambda i: (0, 0)),
        out_shape=jax.ShapeDtypeStruct((1, 1), jnp.float32),
        scratch_shapes=[pltpu.VMEM((1, K), jnp.bfloat16),
                        pltpu.VMEM((K, L), jnp.bfloat16)],
    )(x, a_x, h_x, cluster_centers)
    return out[0, 0]
